# baseline (speedup 1.0000x reference)
import functools

import jax
import jax.numpy as jnp
from jax import lax
from jax.experimental import pallas as pl
from jax.experimental.pallas import tpu as pltpu
from jax.experimental.pallas import tpu_sc as plsc

F32 = jnp.float32
BF16 = jnp.bfloat16


def _dotbf(a, b):
    return jnp.dot(a.astype(BF16), b.astype(BF16), preferred_element_type=F32)

B = 8
HD = 256
K = 1024
HW = 56
TOK = B * HW * HW
VQ_TM = 512
VQ_GRID = TOK // VQ_TM


def _enc1_body(x_ref, w_ref, b_ref, o_ref):
    x = x_ref[0]
    pieces = []
    for di in range(4):
        for dj in range(4):
            ai, ri, aj, rj = di // 2, di % 2, dj // 2, dj % 2
            c0 = (ri * 2 + rj) * 16
            pieces.append(x[ai:ai + 112, aj:aj + 112, c0:c0 + 3])
    p = jnp.concatenate(pieces, axis=2).reshape(112 * 112, 48).astype(BF16)
    acc = jnp.dot(p, w_ref[...].astype(BF16), preferred_element_type=F32)
    y = jnp.maximum(acc + b_ref[0], 0.0)
    o_ref[0] = y.reshape(112, 112, 128)


def _enc2_body(x_ref, w_ref, b_ref, o_ref):
    x = x_ref[0]
    pieces = []
    for di in range(4):
        for dj in range(4):
            ai, ri, aj, rj = di // 2, di % 2, dj // 2, dj % 2
            c0 = (ri * 2 + rj) * 128
            pieces.append(x[ai:ai + HW, aj:aj + HW, c0:c0 + 128].astype(BF16))
    p = jnp.concatenate(pieces, axis=2).reshape(HW * HW, 2048)
    acc = jnp.dot(p, w_ref[...].astype(BF16), preferred_element_type=F32)
    y = jnp.maximum(acc + b_ref[0], 0.0)
    o_ref[0] = jnp.zeros((58, 58, HD), F32)
    o_ref[0, 1:57, 1:57, :] = y.reshape(HW, HW, HD)


def _tap9(x):
    pieces = [x[di:di + HW, dj:dj + HW, :].astype(BF16)
              for di in range(3) for dj in range(3)]
    return jnp.concatenate(pieces, axis=2).reshape(HW * HW, 9 * HD)


def _conv3_body(x_ref, w_ref, b_ref, o_ref):
    x = x_ref[0]
    acc = jnp.dot(_tap9(x), w_ref[...].astype(BF16), preferred_element_type=F32)
    y = acc + b_ref[0]
    o_ref[0] = jnp.zeros((58, 58, HD), F32)
    o_ref[0, 1:57, 1:57, :] = y.reshape(HW, HW, HD)


def _resblock_body(x_ref, w1_ref, b1_ref, w2_ref, b2_ref, o_ref, *, pad_out):
    xp = x_ref[0]
    h = jnp.maximum(xp, 0.0)
    acc = jnp.dot(_tap9(h), w1_ref[...].astype(BF16), preferred_element_type=F32)
    h1 = jnp.maximum(acc + b1_ref[0], 0.0)
    h2 = _dotbf(h1, w2_ref[...]) + b2_ref[0]
    y = xp[1:57, 1:57, :].reshape(HW * HW, HD) + h2
    if pad_out:
        o_ref[0] = jnp.zeros((58, 58, HD), F32)
        o_ref[0, 1:57, 1:57, :] = y.reshape(HW, HW, HD)
    else:
        o_ref[0] = y.reshape(HW, HW, HD)


def _vq_body(z_ref, cbt_ref, cb_ref, idx_ref, csum_ref):
    i = pl.program_id(0)
    z = z_ref[...]
    cbt = cbt_ref[...]
    scores = jnp.dot(z, cbt, preferred_element_type=F32)
    cb = cb_ref[...]
    c2 = jnp.sum(cb * cb, axis=1)
    z2 = jnp.sum(z * z, axis=1)
    d = (z2[:, None] - 2.0 * scores) + c2[None, :]
    m = jnp.min(d, axis=1)
    iota = lax.broadcasted_iota(jnp.int32, (VQ_TM, K), 1).astype(F32)
    idxf = jnp.min(jnp.where(d == m[:, None], iota, F32(K)), axis=1)
    idx_ref[0, 0] = idxf.astype(jnp.int32)
    part = jnp.sum(m)

    @pl.when(i == 0)
    def _init():
        csum_ref[...] = jnp.zeros((1, 1), F32)

    csum_ref[...] += jnp.full((1, 1), part, F32)

    @pl.when(i == VQ_GRID - 1)
    def _fin():
        csum_ref[...] = csum_ref[...] * (1.0 / (TOK * HD))


def _convt1_body(x_ref, w_ref, b_ref, o_ref):
    h = jnp.maximum(x_ref[0], 0.0)
    for ri in range(2):
        for rj in range(2):
            acc = jnp.zeros((HW * HW, 128), F32)
            for ai in range(2):
                for aj in range(2):
                    ht = h[ri + ai:ri + ai + HW, rj + aj:rj + aj + HW, :]
                    acc += _dotbf(ht.reshape(HW * HW, HD),
                                  w_ref[ri, rj, ai, aj])
            y = jnp.maximum(acc + b_ref[0], 0.0)
            o_ref[0, ri, rj] = y.reshape(HW, HW, 128)


def _convt2_body(x_ref, w_ref, b_ref, o_ref):
    p = pl.program_id(1)
    ri = p // 2
    rj = p % 2
    acc = jnp.zeros((112 * 112, 8), F32)
    for ai in range(2):
        for aj in range(2):
            xt = x_ref[0, pl.ds(ri + ai, 112), pl.ds(rj + aj, 112), :]
            acc += _dotbf(xt.reshape(112 * 112, 128),
                          w_ref[ri, rj, ai, aj])
    y = acc + b_ref[0]
    o_ref[0, 0, 0] = y.reshape(112, 112, 8)


_NC, _NS = 2, 16
_GCH = 112


def _sc_gather(cb, idx):
    nw = _NC * _NS
    b_w = TOK // nw
    nchunk = b_w // _GCH
    mesh = plsc.VectorSubcoreMesh(core_axis_name="c", subcore_axis_name="s")

    @functools.partial(
        pl.kernel,
        out_type=jax.ShapeDtypeStruct((TOK, HD), F32),
        mesh=mesh,
        scratch_types=[pltpu.VMEM((_GCH,), jnp.int32),
                       pltpu.VMEM((_GCH, HD), F32),
                       pltpu.SemaphoreType.DMA],
    )
    def gather_k(cb_hbm, idx_hbm, out_hbm, idx_v, rows_v, sem):
        wid = lax.axis_index("s") * _NC + lax.axis_index("c")
        base = wid * b_w
        for c in range(nchunk):
            off = base + c * _GCH
            pltpu.sync_copy(idx_hbm.at[pl.ds(off, _GCH)], idx_v)
            pltpu.async_copy(cb_hbm.at[idx_v], rows_v, sem).wait()
            pltpu.sync_copy(rows_v, out_hbm.at[pl.ds(off, _GCH)])

    return gather_k(cb, idx)


def _full(shape):
    return pl.BlockSpec(shape, lambda *a: (0,) * len(shape))


def _enc1(xs, w, b):
    return pl.pallas_call(
        _enc1_body,
        grid=(B,),
        in_specs=[pl.BlockSpec((1, 113, 113, 64), lambda i: (i, 0, 0, 0)),
                  _full((48, 128)), _full((1, 128))],
        out_specs=pl.BlockSpec((1, 112, 112, 128), lambda i: (i, 0, 0, 0)),
        out_shape=jax.ShapeDtypeStruct((B, 112, 112, 128), F32),
    )(xs, w, b)


def _enc2(xs, w, b):
    return pl.pallas_call(
        _enc2_body,
        grid=(B,),
        in_specs=[pl.BlockSpec((1, 57, 57, 512), lambda i: (i, 0, 0, 0)),
                  _full((2048, 256)), _full((1, 256))],
        out_specs=pl.BlockSpec((1, 58, 58, 256), lambda i: (i, 0, 0, 0)),
        out_shape=jax.ShapeDtypeStruct((B, 58, 58, 256), F32),
    )(xs, w, b)


def _conv3x3(xp, w, b):
    return pl.pallas_call(
        _conv3_body,
        grid=(B,),
        in_specs=[pl.BlockSpec((1, 58, 58, 256), lambda i: (i, 0, 0, 0)),
                  _full((2304, 256)), _full((1, 256))],
        out_specs=pl.BlockSpec((1, 58, 58, 256), lambda i: (i, 0, 0, 0)),
        out_shape=jax.ShapeDtypeStruct((B, 58, 58, 256), F32),
    )(xp, w, b)


def _resblock(xp, w1, b1, w2, b2, pad_out):
    out_s = (B, 58, 58, 256) if pad_out else (B, 56, 56, 256)
    blk = (1,) + out_s[1:]
    return pl.pallas_call(
        functools.partial(_resblock_body, pad_out=pad_out),
        grid=(B,),
        in_specs=[pl.BlockSpec((1, 58, 58, 256), lambda i: (i, 0, 0, 0)),
                  _full((2304, 256)), _full((1, 256)),
                  _full((256, 256)), _full((1, 256))],
        out_specs=pl.BlockSpec(blk, lambda i: (i, 0, 0, 0)),
        out_shape=jax.ShapeDtypeStruct(out_s, F32),
    )(xp, w1, b1, w2, b2)


def _vq(zf, cbt, cb):
    return pl.pallas_call(
        _vq_body,
        grid=(VQ_GRID,),
        in_specs=[pl.BlockSpec((VQ_TM, HD), lambda i: (i, 0)),
                  _full((HD, K)), _full((K, HD))],
        out_specs=[pl.BlockSpec((1, 1, VQ_TM), lambda i: (i, 0, 0)),
                   pl.BlockSpec((1, 1), lambda i: (0, 0))],
        out_shape=[jax.ShapeDtypeStruct((VQ_GRID, 1, VQ_TM), jnp.int32),
                   jax.ShapeDtypeStruct((1, 1), F32)],
    )(zf, cbt, cb)


def _convt1(xp, w, b):
    return pl.pallas_call(
        _convt1_body,
        grid=(B,),
        in_specs=[pl.BlockSpec((1, 58, 58, 256), lambda i: (i, 0, 0, 0)),
                  _full((2, 2, 2, 2, 256, 128)), _full((1, 128))],
        out_specs=pl.BlockSpec((1, 2, 2, 56, 56, 128),
                               lambda i: (i, 0, 0, 0, 0, 0)),
        out_shape=jax.ShapeDtypeStruct((B, 2, 2, 56, 56, 128), F32),
    )(xp, w, b)


def _convt2(xp, w, b):
    return pl.pallas_call(
        _convt2_body,
        grid=(B, 4),
        in_specs=[pl.BlockSpec((1, 114, 114, 128), lambda i, p: (i, 0, 0, 0)),
                  _full((2, 2, 2, 2, 128, 8)), _full((1, 8))],
        out_specs=pl.BlockSpec((1, 1, 1, 112, 112, 8),
                               lambda i, p: (i, p // 2, p % 2, 0, 0, 0)),
        out_shape=jax.ShapeDtypeStruct((B, 2, 2, 112, 112, 8), F32),
    )(xp, w, b)


def _s2d_weights(w, cpad):
    o, c = w.shape[0], w.shape[1]
    w6 = w.reshape(o, c, 2, 2, 2, 2)
    w6 = w6.transpose(2, 4, 3, 5, 1, 0)
    w6 = jnp.pad(w6, ((0, 0),) * 4 + ((0, cpad - c), (0, 0)))
    return w6.reshape(2, 2, 4 * cpad, o)


def _s2d_input(x, cpad):
    b, h, w, c = x.shape
    xp = jnp.pad(x, ((0, 0), (1, 1), (1, 1), (0, cpad - c)))
    h2, w2 = (h + 2) // 2, (w + 2) // 2
    xp = xp.reshape(b, h2, 2, w2, 2, cpad).transpose(0, 1, 3, 2, 4, 5)
    return xp.reshape(b, h2, w2, 4 * cpad)


def _tap_weights_3x3(w):
    return jnp.transpose(w, (2, 3, 1, 0)).reshape(9 * w.shape[1], w.shape[0])


def _tmajor_weights_4x4_pad(w):
    return jnp.transpose(w, (2, 3, 1, 0)).reshape(48, 128)


def _tmajor_weights_4x4(w):
    return jnp.transpose(w, (2, 3, 1, 0)).reshape(16 * w.shape[1], w.shape[0])


def _convt_weights(w, opad=None):
    o, i = w.shape[0], w.shape[1]
    w6 = w.reshape(o, i, 2, 2, 2, 2)
    w6 = w6.transpose(3, 5, 2, 4, 1, 0)
    if opad is not None:
        w6 = jnp.pad(w6, ((0, 0),) * 5 + ((0, opad - o),))
    return w6


def _pad1(x):
    return jnp.pad(x, ((0, 0), (1, 1), (1, 1), (0, 0)))


def kernel(x, enc_w1, enc_b1, enc_w2, enc_b2, enc_w3, enc_b3,
           enc_r1_w1, enc_r1_b1, enc_r1_w2, enc_r1_b2,
           enc_r2_w1, enc_r2_b1, enc_r2_w2, enc_r2_b2,
           codebook,
           dec_w1, dec_b1,
           dec_r1_w1, dec_r1_b1, dec_r1_w2, dec_r1_b2,
           dec_r2_w1, dec_r2_b1, dec_r2_w2, dec_r2_b2,
           dec_wt1, dec_bt1, dec_wt2, dec_bt2):
    r2 = lambda v: v.reshape(1, -1)

    xs = _s2d_input(jnp.transpose(x, (0, 2, 3, 1)), 16)
    h = _enc1(xs, _tmajor_weights_4x4_pad(enc_w1), r2(enc_b1))
    hs = _s2d_input(h, 128)
    h = _enc2(hs, _tmajor_weights_4x4(enc_w2), r2(enc_b2))
    h = _conv3x3(h, _tap_weights_3x3(enc_w3), r2(enc_b3))
    h = _resblock(h, _tap_weights_3x3(enc_r1_w1), r2(enc_r1_b1),
                  enc_r1_w2[:, :, 0, 0].T, r2(enc_r1_b2), pad_out=True)
    z = _resblock(h, _tap_weights_3x3(enc_r2_w1), r2(enc_r2_b1),
                  enc_r2_w2[:, :, 0, 0].T, r2(enc_r2_b2), pad_out=False)

    zf = z.reshape(TOK, HD)
    idx3, csum = _vq(zf, codebook.T, codebook)
    idx = idx3.reshape(TOK)
    commit_loss = csum[0, 0]
    quant = _sc_gather(codebook, idx)

    q = _pad1(quant.reshape(B, HW, HW, HD))
    h = _conv3x3(q, _tap_weights_3x3(dec_w1), r2(dec_b1))
    h = _resblock(h, _tap_weights_3x3(dec_r1_w1), r2(dec_r1_b1),
                  dec_r1_w2[:, :, 0, 0].T, r2(dec_r1_b2), pad_out=True)
    h = _resblock(h, _tap_weights_3x3(dec_r2_w1), r2(dec_r2_b1),
                  dec_r2_w2[:, :, 0, 0].T, r2(dec_r2_b2), pad_out=True)
    y1 = _convt1(h, _convt_weights(dec_wt1), r2(dec_bt1))
    y1 = y1.transpose(0, 3, 1, 4, 2, 5).reshape(B, 112, 112, 128)
    y1 = _pad1(y1)
    bt2 = jnp.pad(dec_bt2, (0, 5))
    y2 = _convt2(y1, _convt_weights(dec_wt2, opad=8), r2(bt2))
    xr = y2[..., :3].transpose(0, 5, 3, 1, 4, 2).reshape(B, 3, 224, 224)

    idx_out = idx.reshape(B, HW, HW)
    return (xr, commit_loss, idx_out)
